# Initial kernel scaffold; baseline (speedup 1.0000x reference)
import jax
import jax.numpy as jnp
from jax import lax
from jax.experimental import pallas as pl
from jax.experimental.pallas import tpu as pltpu
from jax.experimental.pallas import tpu_sc as plsc

N = 10000
E = 320000
D = 128
HID = 256

NC, NS = 2, 16
NW = NC * NS
EPW = E // NW
CH = 80
NCH = EPW // CH
NP = 10240
RPT = NP // NS

_MESH = plsc.VectorSubcoreMesh(core_axis_name="c", subcore_axis_name="s")


def _make_sc_agg(with_cnt: bool):
    out_type = [jax.ShapeDtypeStruct((NC, NP, D), jnp.float32)]
    scratch = [
        pltpu.VMEM_SHARED((NP, D), jnp.float32),
        pltpu.VMEM((CH,), jnp.int32),
        pltpu.VMEM((CH,), jnp.int32),
        pltpu.VMEM((CH, D), jnp.float32),
        pltpu.SemaphoreType.DMA,
    ]
    if with_cnt:
        out_type.append(jax.ShapeDtypeStruct((NC, NP, 16), jnp.float32))
        scratch += [
            pltpu.VMEM_SHARED((NP, 16), jnp.float32),
            pltpu.VMEM((CH, 16), jnp.float32),
        ]

    def body(table, src, dst, zrow, zcnt, ones_h, *rest):
        if with_cnt:
            sums_out, cnts_out, acc, sidx, didx, rows, sem, cacc, ones_v = rest
        else:
            (sums_out, acc, sidx, didx, rows, sem) = rest
        c = lax.axis_index("c")
        s = lax.axis_index("s")
        wid = c * NS + s
        rbase = s * RPT
        pltpu.sync_copy(zrow, acc.at[pl.ds(rbase, RPT)])
        if with_cnt:
            pltpu.sync_copy(zcnt, cacc.at[pl.ds(rbase, RPT)])
            pltpu.sync_copy(ones_h, ones_v)
        plsc.subcore_barrier()

        ebase = wid * EPW

        def step(j, carry):
            off = ebase + j * CH
            pltpu.sync_copy(src.at[pl.ds(off, CH)], sidx)
            pltpu.sync_copy(dst.at[pl.ds(off, CH)], didx)
            pltpu.async_copy(table.at[sidx], rows, sem).wait()
            pltpu.sync_copy(rows, acc.at[didx], add=True)
            if with_cnt:
                pltpu.sync_copy(ones_v, cacc.at[didx], add=True)
            return carry

        lax.fori_loop(0, NCH, step, 0)
        plsc.subcore_barrier()
        pltpu.sync_copy(acc.at[pl.ds(rbase, RPT)],
                        sums_out.at[c, pl.ds(rbase, RPT)])
        if with_cnt:
            pltpu.sync_copy(cacc.at[pl.ds(rbase, RPT)],
                            cnts_out.at[c, pl.ds(rbase, RPT)])

    return pl.kernel(body, out_type=out_type, mesh=_MESH, scratch_types=scratch)


_sc_agg_cnt = _make_sc_agg(True)
_sc_agg = _make_sc_agg(False)


_R = 400


def _tc1_body(x, sums, cnts, w1l, b1, w1r, w2l, h_out, t_out):
    cnt = cnts[0] + cnts[1]
    deg = jnp.maximum(cnt[:, 0:1], 1.0)
    agg = (sums[0] + sums[1]) / deg
    h = agg @ w1l[...] + x[...] @ w1r[...] + b1[...]
    h = jnp.maximum(h, 0.0)
    h_out[...] = h
    t_out[...] = h @ w2l[...]


def _tc2_body(h, sums, cnts, w2r, b2, out):
    cnt = cnts[0] + cnts[1]
    deg = jnp.maximum(cnt[:, 0:1], 1.0)
    agg = (sums[0] + sums[1]) / deg
    out[...] = agg + h[...] @ w2r[...] + b2[...]


_tc1 = pl.pallas_call(
    _tc1_body,
    grid=(N // _R,),
    in_specs=[
        pl.BlockSpec((_R, D), lambda i: (i, 0)),
        pl.BlockSpec((NC, _R, D), lambda i: (0, i, 0)),
        pl.BlockSpec((NC, _R, 16), lambda i: (0, i, 0)),
        pl.BlockSpec((D, HID), lambda i: (0, 0)),
        pl.BlockSpec((1, HID), lambda i: (0, 0)),
        pl.BlockSpec((D, HID), lambda i: (0, 0)),
        pl.BlockSpec((HID, D), lambda i: (0, 0)),
    ],
    out_specs=[
        pl.BlockSpec((_R, HID), lambda i: (i, 0)),
        pl.BlockSpec((_R, D), lambda i: (i, 0)),
    ],
    out_shape=[
        jax.ShapeDtypeStruct((N, HID), jnp.float32),
        jax.ShapeDtypeStruct((N, D), jnp.float32),
    ],
)

_tc2 = pl.pallas_call(
    _tc2_body,
    grid=(N // _R,),
    in_specs=[
        pl.BlockSpec((_R, HID), lambda i: (i, 0)),
        pl.BlockSpec((NC, _R, D), lambda i: (0, i, 0)),
        pl.BlockSpec((NC, _R, 16), lambda i: (0, i, 0)),
        pl.BlockSpec((HID, D), lambda i: (0, 0)),
        pl.BlockSpec((1, D), lambda i: (0, 0)),
    ],
    out_specs=pl.BlockSpec((_R, D), lambda i: (i, 0)),
    out_shape=jax.ShapeDtypeStruct((N, D), jnp.float32),
)


def kernel(x, edge_index, W1_l, b1, W1_r, W2_l, b2, W2_r):
    src = edge_index[0]
    dst = edge_index[1]
    zrow = jnp.zeros((RPT, D), jnp.float32)
    zcnt = jnp.zeros((RPT, 16), jnp.float32)
    ones_h = jnp.ones((CH, 16), jnp.float32)

    sums1, cnts = _sc_agg_cnt(x, src, dst, zrow, zcnt, ones_h)

    h, t = _tc1(x, sums1, cnts, W1_l.T, b1.reshape(1, HID), W1_r.T, W2_l.T)

    (sums2,) = _sc_agg(t, src, dst, zrow, zcnt, ones_h)

    out = _tc2(h, sums2, cnts, W2_r.T, b2.reshape(1, D))
    return out


# speedup vs baseline: 5.7878x; 5.7878x over previous
import jax
import jax.numpy as jnp
from jax import lax
from jax.experimental import pallas as pl
from jax.experimental.pallas import tpu as pltpu
from jax.experimental.pallas import tpu_sc as plsc

N = 10000
E = 320000
D = 128
HID = 256

NC, NS = 2, 16
NW = NC * NS
EPW = E // NW
CH = 80
NCH = EPW // CH
NP = 10240
RPT = NP // NS

_SC_PARAMS = pltpu.CompilerParams(use_tc_tiling_on_sc=False)

_MESH = plsc.VectorSubcoreMesh(
    core_axis_name="c", subcore_axis_name="s", num_cores=NC, num_subcores=NS
)


def _sums_body(table, src, dst, zrow, sums_out, acc, sidx, didx, rows, sem):
    c = lax.axis_index("c")
    s = lax.axis_index("s")
    wid = c * NS + s
    rbase = s * RPT
    pltpu.sync_copy(zrow, acc.at[pl.ds(rbase, RPT)])
    plsc.subcore_barrier()

    ebase = wid * EPW

    def step(j, carry):
        off = ebase + j * CH
        pltpu.sync_copy(src.at[pl.ds(off, CH)], sidx.at[0])
        pltpu.sync_copy(dst.at[pl.ds(off, CH)], didx.at[0])
        pltpu.async_copy(table.at[sidx.at[0]], rows, sem).wait()
        pltpu.sync_copy(rows, acc.at[didx.at[0]], add=True)
        return carry

    lax.fori_loop(0, NCH, step, 0)
    plsc.subcore_barrier()
    obase = c * NP + rbase
    pltpu.sync_copy(acc.at[pl.ds(rbase, RPT)], sums_out.at[pl.ds(obase, RPT)])


_sc_sums = pl.kernel(
    _sums_body,
    out_type=jax.ShapeDtypeStruct((NC * NP, D), jnp.float32),
    mesh=_MESH,
    scratch_types=[
        pltpu.VMEM_SHARED((NP, D), jnp.float32),
        pltpu.VMEM((1, CH), jnp.int32),
        pltpu.VMEM((1, CH), jnp.int32),
        pltpu.VMEM((CH, D), jnp.float32),
        pltpu.SemaphoreType.DMA,
    ],
    compiler_params=_SC_PARAMS,
)


def _cnts_body(dst, zcnt, ones_h, cnts_out, cacc, didx, ones_v):
    c = lax.axis_index("c")
    s = lax.axis_index("s")
    wid = c * NS + s
    rbase = s * RPT
    pltpu.sync_copy(zcnt, cacc.at[pl.ds(rbase, RPT)])
    pltpu.sync_copy(ones_h, ones_v)
    plsc.subcore_barrier()

    ebase = wid * EPW

    def step(j, carry):
        off = ebase + j * CH
        pltpu.sync_copy(dst.at[pl.ds(off, CH)], didx.at[0])
        pltpu.sync_copy(ones_v, cacc.at[didx.at[0]], add=True)
        return carry

    lax.fori_loop(0, NCH, step, 0)
    plsc.subcore_barrier()
    obase = c * NP + rbase
    pltpu.sync_copy(cacc.at[pl.ds(rbase, RPT)], cnts_out.at[pl.ds(obase, RPT)])


_sc_cnts = pl.kernel(
    _cnts_body,
    out_type=jax.ShapeDtypeStruct((NC * NP, 16), jnp.float32),
    mesh=_MESH,
    scratch_types=[
        pltpu.VMEM_SHARED((NP, 16), jnp.float32),
        pltpu.VMEM((1, CH), jnp.int32),
        pltpu.VMEM((CH, 16), jnp.float32),
    ],
    compiler_params=_SC_PARAMS,
)


_R = 400


def _tc1_body(x, sums, cnts, w1l, b1, w1r, w2l, h_out, t_out):
    cnt = cnts[0] + cnts[1]
    deg = jnp.maximum(cnt[:, 0:1], 1.0)
    agg = (sums[0] + sums[1]) / deg
    h = agg @ w1l[...] + x[...] @ w1r[...] + b1[...]
    h = jnp.maximum(h, 0.0)
    h_out[...] = h
    t_out[...] = h @ w2l[...]


def _tc2_body(h, sums, cnts, w2r, b2, out):
    cnt = cnts[0] + cnts[1]
    deg = jnp.maximum(cnt[:, 0:1], 1.0)
    agg = (sums[0] + sums[1]) / deg
    out[...] = agg + h[...] @ w2r[...] + b2[...]


_tc1 = pl.pallas_call(
    _tc1_body,
    grid=(N // _R,),
    in_specs=[
        pl.BlockSpec((_R, D), lambda i: (i, 0)),
        pl.BlockSpec((NC, _R, D), lambda i: (0, i, 0)),
        pl.BlockSpec((NC, _R, 16), lambda i: (0, i, 0)),
        pl.BlockSpec((D, HID), lambda i: (0, 0)),
        pl.BlockSpec((1, HID), lambda i: (0, 0)),
        pl.BlockSpec((D, HID), lambda i: (0, 0)),
        pl.BlockSpec((HID, D), lambda i: (0, 0)),
    ],
    out_specs=[
        pl.BlockSpec((_R, HID), lambda i: (i, 0)),
        pl.BlockSpec((_R, D), lambda i: (i, 0)),
    ],
    out_shape=[
        jax.ShapeDtypeStruct((N, HID), jnp.float32),
        jax.ShapeDtypeStruct((N, D), jnp.float32),
    ],
)

_tc2 = pl.pallas_call(
    _tc2_body,
    grid=(N // _R,),
    in_specs=[
        pl.BlockSpec((_R, HID), lambda i: (i, 0)),
        pl.BlockSpec((NC, _R, D), lambda i: (0, i, 0)),
        pl.BlockSpec((NC, _R, 16), lambda i: (0, i, 0)),
        pl.BlockSpec((HID, D), lambda i: (0, 0)),
        pl.BlockSpec((1, D), lambda i: (0, 0)),
    ],
    out_specs=pl.BlockSpec((_R, D), lambda i: (i, 0)),
    out_shape=jax.ShapeDtypeStruct((N, D), jnp.float32),
)


def kernel(x, edge_index, W1_l, b1, W1_r, W2_l, b2, W2_r):
    src = edge_index[0]
    dst = edge_index[1]
    zrow = jnp.zeros((RPT, D), jnp.float32)
    zcnt = jnp.zeros((RPT, 16), jnp.float32)
    ones_h = jnp.ones((CH, 16), jnp.float32)

    sums1 = _sc_sums(x, src, dst, zrow).reshape(NC, NP, D)
    cnts = _sc_cnts(dst, zcnt, ones_h).reshape(NC, NP, 16)

    h, t = _tc1(x, sums1, cnts, W1_l.T, b1.reshape(1, HID), W1_r.T, W2_l.T)

    sums2 = _sc_sums(t, src, dst, zrow).reshape(NC, NP, D)

    out = _tc2(h, sums2, cnts, W2_r.T, b2.reshape(1, D))
    return out


# speedup vs baseline: 7.7950x; 1.3468x over previous
import jax
import jax.numpy as jnp
from jax import lax
from jax.experimental import pallas as pl
from jax.experimental.pallas import tpu as pltpu
from jax.experimental.pallas import tpu_sc as plsc

N = 10000
E = 320000
D = 128
HID = 256

NC, NS = 2, 16
NW = NC * NS
EPW = E // NW
CH = 80
NCH = EPW // CH
NP = 10240
RPT = NP // NS

_SC_PARAMS = pltpu.CompilerParams(use_tc_tiling_on_sc=False)

_MESH = plsc.VectorSubcoreMesh(
    core_axis_name="c", subcore_axis_name="s", num_cores=NC, num_subcores=NS
)


def _sums_body(table, edges, zrow, sums_out, acc, idx, rows,
               gsem0, gsem1, ssem0, ssem1):
    c = lax.axis_index("c")
    s = lax.axis_index("s")
    wid = c * NS + s
    rbase = s * RPT
    pltpu.sync_copy(zrow, acc.at[pl.ds(rbase, RPT)])
    plsc.subcore_barrier()

    ebase = wid * EPW
    gsem = (gsem0, gsem1)
    ssem = (ssem0, ssem1)

    def load_idx(slot, j):
        pltpu.sync_copy(edges.at[:, pl.ds(ebase + j * CH, CH)], idx.at[slot])

    def start_gather(slot):
        pltpu.async_copy(table.at[idx.at[slot, 0]], rows.at[slot], gsem[slot])

    def wait_gather(slot):
        pltpu.make_async_copy(table.at[idx.at[slot, 0]], rows.at[slot],
                              gsem[slot]).wait()

    def start_scatter(slot):
        pltpu.async_copy(rows.at[slot], acc.at[idx.at[slot, 1]], ssem[slot],
                         add=True)

    def wait_scatter(slot):
        pltpu.make_async_copy(rows.at[slot], acc.at[idx.at[slot, 1]],
                              ssem[slot]).wait()

    load_idx(0, 0)
    start_gather(0)
    load_idx(1, 1)
    start_gather(1)
    wait_gather(0)
    start_scatter(0)

    def step(i, carry):
        j = 2 * i + 1
        wait_gather(1)
        start_scatter(1)
        wait_scatter(0)
        load_idx(0, j + 1)
        start_gather(0)
        wait_gather(0)
        start_scatter(0)
        wait_scatter(1)
        load_idx(1, j + 2)
        start_gather(1)
        return carry

    lax.fori_loop(0, (NCH - 3) // 2, step, 0)
    wait_gather(1)
    start_scatter(1)
    wait_scatter(0)
    load_idx(0, NCH - 1)
    start_gather(0)
    wait_gather(0)
    start_scatter(0)
    wait_scatter(1)
    wait_scatter(0)

    plsc.subcore_barrier()
    obase = c * NP + rbase
    pltpu.sync_copy(acc.at[pl.ds(rbase, RPT)], sums_out.at[pl.ds(obase, RPT)])


_sc_sums = pl.kernel(
    _sums_body,
    out_type=jax.ShapeDtypeStruct((NC * NP, D), jnp.float32),
    mesh=_MESH,
    scratch_types=[
        pltpu.VMEM_SHARED((NP, D), jnp.float32),
        pltpu.VMEM((2, 2, CH), jnp.int32),
        pltpu.VMEM((2, CH, D), jnp.float32),
        pltpu.SemaphoreType.DMA,
        pltpu.SemaphoreType.DMA,
        pltpu.SemaphoreType.DMA,
        pltpu.SemaphoreType.DMA,
    ],
    compiler_params=_SC_PARAMS,
)


def _cnts_body(dst, zcnt, ones_h, cnts_out, cacc, didx, ones_v):
    c = lax.axis_index("c")
    s = lax.axis_index("s")
    wid = c * NS + s
    rbase = s * RPT
    pltpu.sync_copy(zcnt, cacc.at[pl.ds(rbase, RPT)])
    pltpu.sync_copy(ones_h, ones_v)
    plsc.subcore_barrier()

    ebase = wid * EPW

    def step(j, carry):
        off = ebase + j * CH
        pltpu.sync_copy(dst.at[pl.ds(off, CH)], didx.at[0])
        pltpu.sync_copy(ones_v, cacc.at[didx.at[0]], add=True)
        return carry

    lax.fori_loop(0, NCH, step, 0)
    plsc.subcore_barrier()
    obase = c * NP + rbase
    pltpu.sync_copy(cacc.at[pl.ds(rbase, RPT)], cnts_out.at[pl.ds(obase, RPT)])


_sc_cnts = pl.kernel(
    _cnts_body,
    out_type=jax.ShapeDtypeStruct((NC * NP, 16), jnp.float32),
    mesh=_MESH,
    scratch_types=[
        pltpu.VMEM_SHARED((NP, 16), jnp.float32),
        pltpu.VMEM((1, CH), jnp.int32),
        pltpu.VMEM((CH, 16), jnp.float32),
    ],
    compiler_params=_SC_PARAMS,
)


_R = 400


def _tc1_body(x, sums, cnts, w1l, b1, w1r, w2l, h_out, t_out):
    cnt = cnts[0] + cnts[1]
    deg = jnp.maximum(cnt[:, 0:1], 1.0)
    agg = (sums[0] + sums[1]) / deg
    h = agg @ w1l[...] + x[...] @ w1r[...] + b1[...]
    h = jnp.maximum(h, 0.0)
    h_out[...] = h
    t_out[...] = h @ w2l[...]


def _tc2_body(h, sums, cnts, w2r, b2, out):
    cnt = cnts[0] + cnts[1]
    deg = jnp.maximum(cnt[:, 0:1], 1.0)
    agg = (sums[0] + sums[1]) / deg
    out[...] = agg + h[...] @ w2r[...] + b2[...]


_tc1 = pl.pallas_call(
    _tc1_body,
    grid=(N // _R,),
    in_specs=[
        pl.BlockSpec((_R, D), lambda i: (i, 0)),
        pl.BlockSpec((NC, _R, D), lambda i: (0, i, 0)),
        pl.BlockSpec((NC, _R, 16), lambda i: (0, i, 0)),
        pl.BlockSpec((D, HID), lambda i: (0, 0)),
        pl.BlockSpec((1, HID), lambda i: (0, 0)),
        pl.BlockSpec((D, HID), lambda i: (0, 0)),
        pl.BlockSpec((HID, D), lambda i: (0, 0)),
    ],
    out_specs=[
        pl.BlockSpec((_R, HID), lambda i: (i, 0)),
        pl.BlockSpec((_R, D), lambda i: (i, 0)),
    ],
    out_shape=[
        jax.ShapeDtypeStruct((N, HID), jnp.float32),
        jax.ShapeDtypeStruct((N, D), jnp.float32),
    ],
)

_tc2 = pl.pallas_call(
    _tc2_body,
    grid=(N // _R,),
    in_specs=[
        pl.BlockSpec((_R, HID), lambda i: (i, 0)),
        pl.BlockSpec((NC, _R, D), lambda i: (0, i, 0)),
        pl.BlockSpec((NC, _R, 16), lambda i: (0, i, 0)),
        pl.BlockSpec((HID, D), lambda i: (0, 0)),
        pl.BlockSpec((1, D), lambda i: (0, 0)),
    ],
    out_specs=pl.BlockSpec((_R, D), lambda i: (i, 0)),
    out_shape=jax.ShapeDtypeStruct((N, D), jnp.float32),
)


def kernel(x, edge_index, W1_l, b1, W1_r, W2_l, b2, W2_r):
    dst = edge_index[1]
    zrow = jnp.zeros((RPT, D), jnp.float32)
    zcnt = jnp.zeros((RPT, 16), jnp.float32)
    ones_h = jnp.ones((CH, 16), jnp.float32)

    sums1 = _sc_sums(x, edge_index, zrow).reshape(NC, NP, D)
    cnts = _sc_cnts(dst, zcnt, ones_h).reshape(NC, NP, 16)

    h, t = _tc1(x, sums1, cnts, W1_l.T, b1.reshape(1, HID), W1_r.T, W2_l.T)

    sums2 = _sc_sums(t, edge_index, zrow).reshape(NC, NP, D)

    out = _tc2(h, sums2, cnts, W2_r.T, b2.reshape(1, D))
    return out


# speedup vs baseline: 12.5003x; 1.6036x over previous
import jax
import jax.numpy as jnp
from jax import lax
from jax.experimental import pallas as pl
from jax.experimental.pallas import tpu as pltpu
from jax.experimental.pallas import tpu_sc as plsc

N = 10000
E = 320000
D = 128
HID = 256

NC, NS = 2, 16
NW = NC * NS
EPW = E // NW
CH = 80
NCH = EPW // CH
NP = 10240
RPT = NP // NS

_SC_PARAMS = pltpu.CompilerParams(use_tc_tiling_on_sc=False)

_MESH = plsc.VectorSubcoreMesh(
    core_axis_name="c", subcore_axis_name="s", num_cores=NC, num_subcores=NS
)


_SL = 4
_LAG = 2


def _sums_body(table, edges, zrow, sums_out, acc, idx, rows, *sems):
    gsem, ssem = sems[:_SL], sems[_SL:]
    c = lax.axis_index("c")
    s = lax.axis_index("s")
    wid = c * NS + s
    rbase = s * RPT
    pltpu.sync_copy(zrow, acc.at[pl.ds(rbase, RPT)])
    plsc.subcore_barrier()

    ebase = wid * EPW

    def fire_gather(slot, j):
        pltpu.sync_copy(edges.at[:, pl.ds(ebase + j * CH, CH)], idx.at[slot])
        pltpu.async_copy(table.at[idx.at[slot, 0]], rows.at[slot], gsem[slot])

    def wait_gather(slot):
        pltpu.make_async_copy(table.at[idx.at[slot, 0]], rows.at[slot],
                              gsem[slot]).wait()

    def fire_scatter(slot):
        pltpu.async_copy(rows.at[slot], acc.at[idx.at[slot, 1]], ssem[slot],
                         add=True)

    def wait_scatter(slot):
        pltpu.make_async_copy(rows.at[slot], acc.at[idx.at[slot, 1]],
                              ssem[slot]).wait()

    def step(jg, k, reclaim, scatter):
        if reclaim:
            wait_scatter(k)
        fire_gather(k, jg)
        if scatter:
            k2 = (k - _LAG) % _SL
            wait_gather(k2)
            fire_scatter(k2)

    for j in range(_SL):
        step(j, j, False, j >= _LAG)

    def body(i, carry):
        jbase = _SL + _SL * i
        for k in range(_SL):
            step(jbase + k, k, True, True)
        return carry

    lax.fori_loop(0, NCH // _SL - 1, body, 0)

    for jg in range(_SL * (NCH // _SL), NCH):
        step(jg, jg % _SL, True, True)
    for m in range(NCH - _LAG, NCH):
        k2 = m % _SL
        wait_gather(k2)
        fire_scatter(k2)
    for k in range(_SL):
        wait_scatter(k)

    plsc.subcore_barrier()
    obase = c * NP + rbase
    pltpu.sync_copy(acc.at[pl.ds(rbase, RPT)], sums_out.at[pl.ds(obase, RPT)])


_sc_sums = pl.kernel(
    _sums_body,
    out_type=jax.ShapeDtypeStruct((NC * NP, D), jnp.float32),
    mesh=_MESH,
    scratch_types=[
        pltpu.VMEM_SHARED((NP, D), jnp.float32),
        pltpu.VMEM((_SL, 2, CH), jnp.int32),
        pltpu.VMEM((_SL, CH, D), jnp.float32),
    ] + [pltpu.SemaphoreType.DMA] * (2 * _SL),
    compiler_params=_SC_PARAMS,
)


def _cnts_body(dst, zcnt, ones_h, cnts_out, cacc, didx, ones_v):
    c = lax.axis_index("c")
    s = lax.axis_index("s")
    wid = c * NS + s
    rbase = s * RPT
    pltpu.sync_copy(zcnt, cacc.at[pl.ds(rbase, RPT)])
    pltpu.sync_copy(ones_h, ones_v)
    plsc.subcore_barrier()

    ebase = wid * EPW

    def step(j, carry):
        off = ebase + j * CH
        pltpu.sync_copy(dst.at[pl.ds(off, CH)], didx.at[0])
        pltpu.sync_copy(ones_v, cacc.at[didx.at[0]], add=True)
        return carry

    lax.fori_loop(0, NCH, step, 0)
    plsc.subcore_barrier()
    obase = c * NP + rbase
    pltpu.sync_copy(cacc.at[pl.ds(rbase, RPT)], cnts_out.at[pl.ds(obase, RPT)])


_sc_cnts = pl.kernel(
    _cnts_body,
    out_type=jax.ShapeDtypeStruct((NC * NP, 16), jnp.float32),
    mesh=_MESH,
    scratch_types=[
        pltpu.VMEM_SHARED((NP, 16), jnp.float32),
        pltpu.VMEM((1, CH), jnp.int32),
        pltpu.VMEM((CH, 16), jnp.float32),
    ],
    compiler_params=_SC_PARAMS,
)


_R = 400


def _tc1_body(x, sums, cnts, w1l, b1, w1r, w2l, h_out, t_out):
    cnt = cnts[0] + cnts[1]
    deg = jnp.maximum(cnt[:, 0:1], 1.0)
    agg = (sums[0] + sums[1]) / deg
    h = agg @ w1l[...] + x[...] @ w1r[...] + b1[...]
    h = jnp.maximum(h, 0.0)
    h_out[...] = h
    t_out[...] = h @ w2l[...]


def _tc2_body(h, sums, cnts, w2r, b2, out):
    cnt = cnts[0] + cnts[1]
    deg = jnp.maximum(cnt[:, 0:1], 1.0)
    agg = (sums[0] + sums[1]) / deg
    out[...] = agg + h[...] @ w2r[...] + b2[...]


_tc1 = pl.pallas_call(
    _tc1_body,
    grid=(N // _R,),
    in_specs=[
        pl.BlockSpec((_R, D), lambda i: (i, 0)),
        pl.BlockSpec((NC, _R, D), lambda i: (0, i, 0)),
        pl.BlockSpec((NC, _R, 16), lambda i: (0, i, 0)),
        pl.BlockSpec((D, HID), lambda i: (0, 0)),
        pl.BlockSpec((1, HID), lambda i: (0, 0)),
        pl.BlockSpec((D, HID), lambda i: (0, 0)),
        pl.BlockSpec((HID, D), lambda i: (0, 0)),
    ],
    out_specs=[
        pl.BlockSpec((_R, HID), lambda i: (i, 0)),
        pl.BlockSpec((_R, D), lambda i: (i, 0)),
    ],
    out_shape=[
        jax.ShapeDtypeStruct((N, HID), jnp.float32),
        jax.ShapeDtypeStruct((N, D), jnp.float32),
    ],
)

_tc2 = pl.pallas_call(
    _tc2_body,
    grid=(N // _R,),
    in_specs=[
        pl.BlockSpec((_R, HID), lambda i: (i, 0)),
        pl.BlockSpec((NC, _R, D), lambda i: (0, i, 0)),
        pl.BlockSpec((NC, _R, 16), lambda i: (0, i, 0)),
        pl.BlockSpec((HID, D), lambda i: (0, 0)),
        pl.BlockSpec((1, D), lambda i: (0, 0)),
    ],
    out_specs=pl.BlockSpec((_R, D), lambda i: (i, 0)),
    out_shape=jax.ShapeDtypeStruct((N, D), jnp.float32),
)


def kernel(x, edge_index, W1_l, b1, W1_r, W2_l, b2, W2_r):
    dst = edge_index[1]
    zrow = jnp.zeros((RPT, D), jnp.float32)
    zcnt = jnp.zeros((RPT, 16), jnp.float32)
    ones_h = jnp.ones((CH, 16), jnp.float32)

    sums1 = _sc_sums(x, edge_index, zrow).reshape(NC, NP, D)
    cnts = _sc_cnts(dst, zcnt, ones_h).reshape(NC, NP, 16)

    h, t = _tc1(x, sums1, cnts, W1_l.T, b1.reshape(1, HID), W1_r.T, W2_l.T)

    sums2 = _sc_sums(t, edge_index, zrow).reshape(NC, NP, D)

    out = _tc2(h, sums2, cnts, W2_r.T, b2.reshape(1, D))
    return out


# speedup vs baseline: 13.4872x; 1.0789x over previous
import jax
import jax.numpy as jnp
from jax import lax
from jax.experimental import pallas as pl
from jax.experimental.pallas import tpu as pltpu
from jax.experimental.pallas import tpu_sc as plsc

N = 10000
E = 320000
D = 128
HID = 256

NC, NS = 2, 16
NW = NC * NS
EPW = E // NW
CH = 80
NCH = EPW // CH
NP = 10240
RPT = NP // NS

_SC_PARAMS = pltpu.CompilerParams(use_tc_tiling_on_sc=False)

_MESH = plsc.VectorSubcoreMesh(
    core_axis_name="c", subcore_axis_name="s", num_cores=NC, num_subcores=NS
)


_SL = 4
_LAG = 2


def _sums_body(table, edges, zrow, sums_out, acc, idx, rows, *sems):
    gsem, ssem = sems[:_SL], sems[_SL:]
    c = lax.axis_index("c")
    s = lax.axis_index("s")
    wid = c * NS + s
    rbase = s * RPT
    pltpu.sync_copy(zrow, acc.at[pl.ds(rbase, RPT)])
    plsc.subcore_barrier()

    ebase = wid * EPW

    def fire_gather(slot, j):
        pltpu.sync_copy(edges.at[:, pl.ds(ebase + j * CH, CH)], idx.at[slot])
        pltpu.async_copy(table.at[idx.at[slot, 0]], rows.at[slot], gsem[slot])

    def wait_gather(slot):
        pltpu.make_async_copy(table.at[idx.at[slot, 0]], rows.at[slot],
                              gsem[slot]).wait()

    def fire_scatter(slot):
        pltpu.async_copy(rows.at[slot], acc.at[idx.at[slot, 1]], ssem[slot],
                         add=True)

    def wait_scatter(slot):
        pltpu.make_async_copy(rows.at[slot], acc.at[idx.at[slot, 1]],
                              ssem[slot]).wait()

    def step(jg, k, reclaim, scatter):
        if reclaim:
            wait_scatter(k)
        fire_gather(k, jg)
        if scatter:
            k2 = (k - _LAG) % _SL
            wait_gather(k2)
            fire_scatter(k2)

    for j in range(_SL):
        step(j, j, False, j >= _LAG)

    def body(i, carry):
        jbase = _SL + _SL * i
        for k in range(_SL):
            step(jbase + k, k, True, True)
        return carry

    lax.fori_loop(0, NCH // _SL - 1, body, 0)

    for jg in range(_SL * (NCH // _SL), NCH):
        step(jg, jg % _SL, True, True)
    for m in range(NCH - _LAG, NCH):
        k2 = m % _SL
        wait_gather(k2)
        fire_scatter(k2)
    for k in range(_SL):
        wait_scatter(k)

    plsc.subcore_barrier()
    obase = c * NP + rbase
    pltpu.sync_copy(acc.at[pl.ds(rbase, RPT)], sums_out.at[pl.ds(obase, RPT)])


_sc_sums = pl.kernel(
    _sums_body,
    out_type=jax.ShapeDtypeStruct((NC * NP, D), jnp.float32),
    mesh=_MESH,
    scratch_types=[
        pltpu.VMEM_SHARED((NP, D), jnp.float32),
        pltpu.VMEM((_SL, 2, CH), jnp.int32),
        pltpu.VMEM((_SL, CH, D), jnp.float32),
    ] + [pltpu.SemaphoreType.DMA] * (2 * _SL),
    compiler_params=_SC_PARAMS,
)


def _cnts_body(edges, zcnt, ones_h, cnts_out, cacc, didx, ones_v, *ssem):
    c = lax.axis_index("c")
    s = lax.axis_index("s")
    wid = c * NS + s
    rbase = s * RPT
    pltpu.sync_copy(zcnt, cacc.at[pl.ds(rbase, RPT)])
    pltpu.sync_copy(ones_h, ones_v)
    plsc.subcore_barrier()

    ebase = wid * EPW

    def step(j, k, reclaim):
        if reclaim:
            pltpu.make_async_copy(ones_v, cacc.at[didx.at[k]], ssem[k]).wait()
        pltpu.sync_copy(edges.at[1, pl.ds(ebase + j * CH, CH)], didx.at[k])
        pltpu.async_copy(ones_v, cacc.at[didx.at[k]], ssem[k], add=True)

    for j in range(_SL):
        step(j, j, False)

    def body(i, carry):
        jbase = _SL + _SL * i
        for k in range(_SL):
            step(jbase + k, k, True)
        return carry

    lax.fori_loop(0, NCH // _SL - 1, body, 0)
    for jg in range(_SL * (NCH // _SL), NCH):
        step(jg, jg % _SL, True)
    for k in range(_SL):
        pltpu.make_async_copy(ones_v, cacc.at[didx.at[k]], ssem[k]).wait()

    plsc.subcore_barrier()
    obase = c * NP + rbase
    pltpu.sync_copy(cacc.at[pl.ds(rbase, RPT)], cnts_out.at[pl.ds(obase, RPT)])


_sc_cnts = pl.kernel(
    _cnts_body,
    out_type=jax.ShapeDtypeStruct((NC * NP, 16), jnp.float32),
    mesh=_MESH,
    scratch_types=[
        pltpu.VMEM_SHARED((NP, 16), jnp.float32),
        pltpu.VMEM((_SL, CH), jnp.int32),
        pltpu.VMEM((CH, 16), jnp.float32),
    ] + [pltpu.SemaphoreType.DMA] * _SL,
    compiler_params=_SC_PARAMS,
)


_R = 400


def _tc1_body(x, sums, cnts, w1l, b1, w1r, w2l, h_out, t_out):
    cnt = cnts[0] + cnts[1]
    deg = jnp.maximum(cnt[:, 0:1], 1.0)
    agg = (sums[0] + sums[1]) / deg
    h = agg @ w1l[...] + x[...] @ w1r[...] + b1[...]
    h = jnp.maximum(h, 0.0)
    h_out[...] = h
    t_out[...] = h @ w2l[...]


def _tc2_body(h, sums, cnts, w2r, b2, out):
    cnt = cnts[0] + cnts[1]
    deg = jnp.maximum(cnt[:, 0:1], 1.0)
    agg = (sums[0] + sums[1]) / deg
    out[...] = agg + h[...] @ w2r[...] + b2[...]


_tc1 = pl.pallas_call(
    _tc1_body,
    grid=(N // _R,),
    in_specs=[
        pl.BlockSpec((_R, D), lambda i: (i, 0)),
        pl.BlockSpec((NC, _R, D), lambda i: (0, i, 0)),
        pl.BlockSpec((NC, _R, 16), lambda i: (0, i, 0)),
        pl.BlockSpec((D, HID), lambda i: (0, 0)),
        pl.BlockSpec((1, HID), lambda i: (0, 0)),
        pl.BlockSpec((D, HID), lambda i: (0, 0)),
        pl.BlockSpec((HID, D), lambda i: (0, 0)),
    ],
    out_specs=[
        pl.BlockSpec((_R, HID), lambda i: (i, 0)),
        pl.BlockSpec((_R, D), lambda i: (i, 0)),
    ],
    out_shape=[
        jax.ShapeDtypeStruct((N, HID), jnp.float32),
        jax.ShapeDtypeStruct((N, D), jnp.float32),
    ],
)

_tc2 = pl.pallas_call(
    _tc2_body,
    grid=(N // _R,),
    in_specs=[
        pl.BlockSpec((_R, HID), lambda i: (i, 0)),
        pl.BlockSpec((NC, _R, D), lambda i: (0, i, 0)),
        pl.BlockSpec((NC, _R, 16), lambda i: (0, i, 0)),
        pl.BlockSpec((HID, D), lambda i: (0, 0)),
        pl.BlockSpec((1, D), lambda i: (0, 0)),
    ],
    out_specs=pl.BlockSpec((_R, D), lambda i: (i, 0)),
    out_shape=jax.ShapeDtypeStruct((N, D), jnp.float32),
)


def kernel(x, edge_index, W1_l, b1, W1_r, W2_l, b2, W2_r):
    zrow = jnp.zeros((RPT, D), jnp.float32)
    zcnt = jnp.zeros((RPT, 16), jnp.float32)
    ones_h = jnp.ones((CH, 16), jnp.float32)

    sums1 = _sc_sums(x, edge_index, zrow).reshape(NC, NP, D)
    cnts = _sc_cnts(edge_index, zcnt, ones_h).reshape(NC, NP, 16)

    h, t = _tc1(x, sums1, cnts, W1_l.T, b1.reshape(1, HID), W1_r.T, W2_l.T)

    sums2 = _sc_sums(t, edge_index, zrow).reshape(NC, NP, D)

    out = _tc2(h, sums2, cnts, W2_r.T, b2.reshape(1, D))
    return out


# speedup vs baseline: 15.0003x; 1.1122x over previous
import jax
import jax.numpy as jnp
from jax import lax
from jax.experimental import pallas as pl
from jax.experimental.pallas import tpu as pltpu
from jax.experimental.pallas import tpu_sc as plsc

N = 10000
E = 320000
D = 128
HID = 256

NC, NS = 2, 16
NW = NC * NS
EPW = E // NW
CH = 80
NCH = EPW // CH
NP = 10240
RPT = NP // NS

_SC_PARAMS = pltpu.CompilerParams(use_tc_tiling_on_sc=False)

_MESH = plsc.VectorSubcoreMesh(
    core_axis_name="c", subcore_axis_name="s", num_cores=NC, num_subcores=NS
)


_SL = 4
_LAG = 2


def _sums_body(table, edges, zrow, sums_out, acc, idx, rows, *sems):
    gsem, ssem = sems[:_SL], sems[_SL:]
    c = lax.axis_index("c")
    s = lax.axis_index("s")
    wid = c * NS + s
    rbase = s * RPT
    pltpu.sync_copy(zrow, acc.at[pl.ds(rbase, RPT)])
    plsc.subcore_barrier()

    ebase = wid * EPW

    def fire_gather(slot, j):
        pltpu.sync_copy(edges.at[:, pl.ds(ebase + j * CH, CH)], idx.at[slot])
        pltpu.async_copy(table.at[idx.at[slot, 0]], rows.at[slot], gsem[slot])

    def wait_gather(slot):
        pltpu.make_async_copy(table.at[idx.at[slot, 0]], rows.at[slot],
                              gsem[slot]).wait()

    def fire_scatter(slot):
        pltpu.async_copy(rows.at[slot], acc.at[idx.at[slot, 1]], ssem[slot],
                         add=True)

    def wait_scatter(slot):
        pltpu.make_async_copy(rows.at[slot], acc.at[idx.at[slot, 1]],
                              ssem[slot]).wait()

    def step(jg, k, reclaim, scatter):
        if reclaim:
            wait_scatter(k)
        fire_gather(k, jg)
        if scatter:
            k2 = (k - _LAG) % _SL
            wait_gather(k2)
            fire_scatter(k2)

    for j in range(_SL):
        step(j, j, False, j >= _LAG)

    def body(i, carry):
        jbase = _SL + _SL * i
        for k in range(_SL):
            step(jbase + k, k, True, True)
        return carry

    lax.fori_loop(0, NCH // _SL - 1, body, 0)

    for jg in range(_SL * (NCH // _SL), NCH):
        step(jg, jg % _SL, True, True)
    for m in range(NCH - _LAG, NCH):
        k2 = m % _SL
        wait_gather(k2)
        fire_scatter(k2)
    for k in range(_SL):
        wait_scatter(k)

    plsc.subcore_barrier()
    obase = c * NP + rbase
    pltpu.sync_copy(acc.at[pl.ds(rbase, RPT)], sums_out.at[pl.ds(obase, RPT)])


_sc_sums = pl.kernel(
    _sums_body,
    out_type=jax.ShapeDtypeStruct((NC * NP, D), jnp.float32),
    mesh=_MESH,
    scratch_types=[
        pltpu.VMEM_SHARED((NP, D), jnp.float32),
        pltpu.VMEM((_SL, 2, CH), jnp.int32),
        pltpu.VMEM((_SL, CH, D), jnp.float32),
    ] + [pltpu.SemaphoreType.DMA] * (2 * _SL),
    compiler_params=_SC_PARAMS,
)


def _cnts_body(edges, zcnt, ones_h, cnts_out, cacc, didx, ones_v, *ssem):
    c = lax.axis_index("c")
    s = lax.axis_index("s")
    wid = c * NS + s
    rbase = s * RPT
    pltpu.sync_copy(zcnt, cacc.at[pl.ds(rbase, RPT)])
    pltpu.sync_copy(ones_h, ones_v)
    plsc.subcore_barrier()

    ebase = wid * EPW

    isem, ssem = ssem[:_SL], ssem[_SL:]

    def fire_iload(k, j):
        pltpu.async_copy(edges.at[1, pl.ds(ebase + j * CH, CH)], didx.at[k],
                         isem[k])

    def step(j, k, reclaim, scatter):
        if reclaim:
            pltpu.make_async_copy(ones_v, cacc.at[didx.at[k]], ssem[k]).wait()
        fire_iload(k, j)
        if scatter:
            k2 = (k - _LAG) % _SL
            pltpu.make_async_copy(edges.at[1, pl.ds(0, CH)], didx.at[k2],
                                  isem[k2]).wait()
            pltpu.async_copy(ones_v, cacc.at[didx.at[k2]], ssem[k2], add=True)

    for j in range(_SL):
        step(j, j, False, j >= _LAG)

    def body(i, carry):
        jbase = _SL + _SL * i
        for k in range(_SL):
            step(jbase + k, k, True, True)
        return carry

    lax.fori_loop(0, NCH // _SL - 1, body, 0)
    for jg in range(_SL * (NCH // _SL), NCH):
        step(jg, jg % _SL, True, True)
    for m in range(NCH - _LAG, NCH):
        k2 = m % _SL
        pltpu.make_async_copy(edges.at[1, pl.ds(0, CH)], didx.at[k2],
                              isem[k2]).wait()
        pltpu.async_copy(ones_v, cacc.at[didx.at[k2]], ssem[k2], add=True)
    for k in range(_SL):
        pltpu.make_async_copy(ones_v, cacc.at[didx.at[k]], ssem[k]).wait()

    plsc.subcore_barrier()
    obase = c * NP + rbase
    pltpu.sync_copy(cacc.at[pl.ds(rbase, RPT)], cnts_out.at[pl.ds(obase, RPT)])


_sc_cnts = pl.kernel(
    _cnts_body,
    out_type=jax.ShapeDtypeStruct((NC * NP, 16), jnp.float32),
    mesh=_MESH,
    scratch_types=[
        pltpu.VMEM_SHARED((NP, 16), jnp.float32),
        pltpu.VMEM((_SL, CH), jnp.int32),
        pltpu.VMEM((CH, 16), jnp.float32),
    ] + [pltpu.SemaphoreType.DMA] * (2 * _SL),
    compiler_params=_SC_PARAMS,
)


_R = 400


def _tc1_body(x, sums, cnts, w1l, b1, w1r, w2l, h_out, t_out):
    cnt = cnts[0] + cnts[1]
    deg = jnp.maximum(cnt[:, 0:1], 1.0)
    agg = (sums[0] + sums[1]) / deg
    h = agg @ w1l[...] + x[...] @ w1r[...] + b1[...]
    h = jnp.maximum(h, 0.0)
    h_out[...] = h
    t_out[...] = h @ w2l[...]


def _tc2_body(h, sums, cnts, w2r, b2, out):
    cnt = cnts[0] + cnts[1]
    deg = jnp.maximum(cnt[:, 0:1], 1.0)
    agg = (sums[0] + sums[1]) / deg
    out[...] = agg + h[...] @ w2r[...] + b2[...]


_tc1 = pl.pallas_call(
    _tc1_body,
    grid=(N // _R,),
    in_specs=[
        pl.BlockSpec((_R, D), lambda i: (i, 0)),
        pl.BlockSpec((NC, _R, D), lambda i: (0, i, 0)),
        pl.BlockSpec((NC, _R, 16), lambda i: (0, i, 0)),
        pl.BlockSpec((D, HID), lambda i: (0, 0)),
        pl.BlockSpec((1, HID), lambda i: (0, 0)),
        pl.BlockSpec((D, HID), lambda i: (0, 0)),
        pl.BlockSpec((HID, D), lambda i: (0, 0)),
    ],
    out_specs=[
        pl.BlockSpec((_R, HID), lambda i: (i, 0)),
        pl.BlockSpec((_R, D), lambda i: (i, 0)),
    ],
    out_shape=[
        jax.ShapeDtypeStruct((N, HID), jnp.float32),
        jax.ShapeDtypeStruct((N, D), jnp.float32),
    ],
)

_tc2 = pl.pallas_call(
    _tc2_body,
    grid=(N // _R,),
    in_specs=[
        pl.BlockSpec((_R, HID), lambda i: (i, 0)),
        pl.BlockSpec((NC, _R, D), lambda i: (0, i, 0)),
        pl.BlockSpec((NC, _R, 16), lambda i: (0, i, 0)),
        pl.BlockSpec((HID, D), lambda i: (0, 0)),
        pl.BlockSpec((1, D), lambda i: (0, 0)),
    ],
    out_specs=pl.BlockSpec((_R, D), lambda i: (i, 0)),
    out_shape=jax.ShapeDtypeStruct((N, D), jnp.float32),
)


def kernel(x, edge_index, W1_l, b1, W1_r, W2_l, b2, W2_r):
    zrow = jnp.zeros((RPT, D), jnp.float32)
    zcnt = jnp.zeros((RPT, 16), jnp.float32)
    ones_h = jnp.ones((CH, 16), jnp.float32)

    sums1 = _sc_sums(x, edge_index, zrow).reshape(NC, NP, D)
    cnts = _sc_cnts(edge_index, zcnt, ones_h).reshape(NC, NP, 16)

    h, t = _tc1(x, sums1, cnts, W1_l.T, b1.reshape(1, HID), W1_r.T, W2_l.T)

    sums2 = _sc_sums(t, edge_index, zrow).reshape(NC, NP, D)

    out = _tc2(h, sums2, cnts, W2_r.T, b2.reshape(1, D))
    return out
